# baseline (speedup 1.0000x reference)
import dataclasses
import functools

import numpy as np
import jax
import jax.numpy as jnp
from jax import lax
from jax.experimental import pallas as pl
from jax.experimental.pallas import tpu as pltpu
from jax.experimental.pallas import tpu_sc as plsc

_NUM_FIELDS = 26
_EMBED_DIM = 16
_EMBED_OUT = _NUM_FIELDS * _EMBED_DIM
_B = 16384
_VOCAB = 100000 * _NUM_FIELDS
_N_E = _EMBED_OUT * _B
_N_L = _NUM_FIELDS * _B
_OFFS = np.arange(_NUM_FIELDS, dtype=np.int32) * 100000

_NC = 2
_NS = 16
_NW = _NC * _NS
_EPW = _N_E // _NW
_LPW = _N_L // _NW
_CHUNK = 4096
_LCHUNK = 3328

_S_MAT = np.tile(np.eye(_EMBED_DIM, dtype=np.float32), (_NUM_FIELDS, 1))


def _sc_compiler_params():
    cp = pltpu.CompilerParams(use_tc_tiling_on_sc=False)
    if "needs_layout_passes" in pltpu.CompilerParams.__dataclass_fields__:
        cp = dataclasses.replace(cp, needs_layout_passes=False)
    return cp


@functools.lru_cache(maxsize=1)
def _build_sc_gather():
    @functools.partial(
        pl.kernel,
        out_type=[
            jax.ShapeDtypeStruct((_N_E,), jnp.float32),
            jax.ShapeDtypeStruct((_N_L,), jnp.float32),
        ],
        mesh=plsc.VectorSubcoreMesh(core_axis_name="c", subcore_axis_name="s"),
        scratch_types=[
            pltpu.VMEM((_CHUNK,), jnp.int32),
            pltpu.VMEM((_CHUNK,), jnp.float32),
            pltpu.VMEM((_LCHUNK,), jnp.int32),
            pltpu.VMEM((_LCHUNK,), jnp.float32),
            pltpu.SemaphoreType.DMA,
        ],
        compiler_params=_sc_compiler_params(),
    )
    def _sc_gather(eidx_hbm, lidx_hbm, et_flat, l_flat,
                   gt_out, lval_out,
                   eidx_v, eval_v, lidx_v, lval_v, sem):
        wid = lax.axis_index("s") * _NC + lax.axis_index("c")
        ebase = wid * _EPW
        lbase = wid * _LPW

        @pl.loop(0, _EPW, step=_CHUNK)
        def _(off):
            start = ebase + off
            pltpu.sync_copy(eidx_hbm.at[pl.ds(start, _CHUNK)], eidx_v)
            pltpu.async_copy(et_flat.at[eidx_v], eval_v, sem).wait()
            pltpu.sync_copy(eval_v, gt_out.at[pl.ds(start, _CHUNK)])

        @pl.loop(0, _LPW, step=_LCHUNK)
        def _(off):
            start = lbase + off
            pltpu.sync_copy(lidx_hbm.at[pl.ds(start, _LCHUNK)], lidx_v)
            pltpu.async_copy(l_flat.at[lidx_v], lval_v, sem).wait()
            pltpu.sync_copy(lval_v, lval_out.at[pl.ds(start, _LCHUNK)])

    return _sc_gather


_BLK = 2048
_NB = _B // _BLK
_CONTRACT0 = (((0,), (0,)), ((), ()))


def _tc_body(gt_ref, lval_ref, W1_ref, b1_ref, g1_ref, be1_ref,
             W2_ref, b2_ref, g2_ref, be2_ref, W3_ref, sc_ref, S_ref,
             out_ref, h1_s, base_s):
    i = pl.program_id(0)
    M = gt_ref[...]
    h1 = lax.dot_general(M, W1_ref[...], _CONTRACT0,
                         preferred_element_type=jnp.float32) + b1_ref[...]
    h1_s[pl.ds(i * _BLK, _BLK), :] = h1

    s = lax.dot_general(M, S_ref[...], _CONTRACT0,
                        preferred_element_type=jnp.float32)
    fm = 0.5 * (jnp.sum(s * s, axis=1) - jnp.sum(M * M, axis=0))
    lin = jnp.sum(lval_ref[...], axis=0)
    base_s[pl.ds(i * _BLK, _BLK)] = lin + fm + sc_ref[0]

    @pl.when(i == _NB - 1)
    def _():
        H1 = h1_s[...]
        mu1 = jnp.mean(H1, axis=0, keepdims=True)
        d1 = H1 - mu1
        var1 = jnp.mean(d1 * d1, axis=0, keepdims=True)
        a1 = g1_ref[...] * lax.rsqrt(var1 + 1e-5)
        N1 = jnp.maximum(d1 * a1 + be1_ref[...], 0.0)
        H2 = jnp.dot(N1, W2_ref[...], preferred_element_type=jnp.float32) + b2_ref[...]
        mu2 = jnp.mean(H2, axis=0, keepdims=True)
        d2 = H2 - mu2
        var2 = jnp.mean(d2 * d2, axis=0, keepdims=True)
        a2 = g2_ref[...] * lax.rsqrt(var2 + 1e-5)
        N2 = jnp.maximum(d2 * a2 + be2_ref[...], 0.0)
        mlp = jnp.dot(N2, W3_ref[...], preferred_element_type=jnp.float32)[:, 0]
        z = base_s[...] + mlp
        e = jnp.exp(-jnp.abs(z))
        out_ref[...] = jnp.where(z >= 0, 1.0 / (1.0 + e), e / (1.0 + e))


def _tc_mlp(GT, lvalT, W1, b1, g1, be1, W2, b2, g2, be2, W3, sc):
    full = lambda shape: pl.BlockSpec(shape, lambda i: tuple(0 for _ in shape))
    return pl.pallas_call(
        _tc_body,
        grid=(_NB,),
        in_specs=[
            pl.BlockSpec((_EMBED_OUT, _BLK), lambda i: (0, i)),
            pl.BlockSpec((_NUM_FIELDS, _BLK), lambda i: (0, i)),
            full((_EMBED_OUT, 128)),
            full((1, 128)),
            full((1, 128)),
            full((1, 128)),
            full((128, 128)),
            full((1, 128)),
            full((1, 128)),
            full((1, 128)),
            full((128, 1)),
            pl.BlockSpec(memory_space=pltpu.SMEM),
            full((_EMBED_OUT, _EMBED_DIM)),
        ],
        out_specs=pl.BlockSpec((_B,), lambda i: (0,)),
        out_shape=jax.ShapeDtypeStruct((_B,), jnp.float32),
        scratch_shapes=[
            pltpu.VMEM((_B, 128), jnp.float32),
            pltpu.VMEM((_B,), jnp.float32),
        ],
    )(GT, lvalT, W1, b1.reshape(1, 128), g1.reshape(1, 128), be1.reshape(1, 128),
      W2, b2.reshape(1, 128), g2.reshape(1, 128), be2.reshape(1, 128),
      W3, sc, jnp.asarray(_S_MAT))


def kernel(x, E, L, bias, W1, b1, g1, be1, W2, b2, g2, be2, W3, b3):
    idxT = x.T.astype(jnp.int32) + jnp.asarray(_OFFS)[:, None]
    eidx = (idxT[:, None, :]
            + (jnp.arange(_EMBED_DIM, dtype=jnp.int32) * _VOCAB)[None, :, None])
    eidx = eidx.reshape(_N_E)
    lidx = idxT.reshape(_N_L)
    et_flat = E.T.reshape(_EMBED_DIM * _VOCAB)
    l_flat = L.reshape(_VOCAB)
    gt_flat, lval_flat = _build_sc_gather()(eidx, lidx, et_flat, l_flat)
    GT = gt_flat.reshape(_EMBED_OUT, _B)
    lvalT = lval_flat.reshape(_NUM_FIELDS, _B)
    sc = (bias + b3).reshape(1)
    return _tc_mlp(GT, lvalT, W1, b1, g1, be1, W2, b2, g2, be2, W3, sc)
